# baseline (speedup 1.0000x reference)
import jax
import jax.numpy as jnp
from jax.experimental import pallas as pl
from jax.experimental.pallas import tpu as pltpu

_NUM_CLASSES = 3
_NUM_ANC = 5
_B, _H, _W = 128, 52, 52
_HW = _H * _W
_N = _HW * _NUM_ANC
_CH = 1 + _NUM_CLASSES + 4
_IB = 2


def _bce(logits, targets):
    return (jnp.maximum(logits, 0.0) - logits * targets
            + jnp.log1p(jnp.exp(-jnp.abs(logits))))


def _loss_kernel(p_ref, g_ref, out_ref, acc):
    step = pl.program_id(0)

    @pl.when(step == 0)
    def _init():
        for k in range(6):
            acc[k] = 0.0

    gc = g_ref[:, 0, :]
    mp = (gc > 0.5).astype(jnp.float32)
    mn = (jnp.abs(gc) < 0.5).astype(jnp.float32)
    pc = jax.nn.sigmoid(p_ref[:, 0, :])
    dv = pc - gc
    acc[0] += jnp.sum(dv * dv * (5.0 * mp + mn))
    acc[4] += jnp.sum(mp)

    t2 = 0.0
    for j in range(1, 1 + _NUM_CLASSES):
        t2 += jnp.sum(_bce(p_ref[:, j, :], g_ref[:, j, :]) * mp)
    acc[1] += t2

    mw = mp * g_ref[:, 8, :]
    t3 = 0.0
    for j in (4, 5):
        t3 += jnp.sum(_bce(p_ref[:, j, :], g_ref[:, j, :]) * mw)
    acc[2] += t3

    t4 = 0.0
    for j in (6, 7):
        d = p_ref[:, j, :] - g_ref[:, j, :]
        t4 += jnp.sum(d * d * mw)
    acc[3] += t4

    @pl.when(step == pl.num_programs(0) - 1)
    def _fin():
        npos = jnp.maximum(acc[4], 1.0)
        out_ref[0] = ((acc[0] + acc[2] + acc[3]) / float(_B)
                      + acc[1] / npos)


def kernel(pyolos, gyolos):
    pn = (pyolos.reshape(_B, _CH, _NUM_ANC, _HW)
          .transpose(0, 1, 3, 2)
          .reshape(_B, _CH, _N))
    gt = jnp.transpose(gyolos, (0, 2, 1))
    out = pl.pallas_call(
        _loss_kernel,
        grid=(_B // _IB,),
        in_specs=[
            pl.BlockSpec((_IB, _CH, _N), lambda i: (i, 0, 0)),
            pl.BlockSpec((_IB, 13, _N), lambda i: (i, 0, 0)),
        ],
        out_specs=pl.BlockSpec(memory_space=pltpu.SMEM),
        out_shape=jax.ShapeDtypeStruct((1,), jnp.float32),
        scratch_shapes=[pltpu.SMEM((8,), jnp.float32)],
        compiler_params=pltpu.CompilerParams(
            dimension_semantics=("arbitrary",)),
    )(pn, gt)
    return out[0]


# speedup vs baseline: 1.0566x; 1.0566x over previous
import jax
import jax.numpy as jnp
from jax.experimental import pallas as pl
from jax.experimental.pallas import tpu as pltpu

_NUM_CLASSES = 3
_NUM_ANC = 5
_B, _H, _W = 128, 52, 52
_HW = _H * _W
_N = _HW * _NUM_ANC
_CH = 1 + _NUM_CLASSES + 4
_IB = 4


def _bce(logits, targets):
    return (jnp.maximum(logits, 0.0) - logits * targets
            + jnp.log1p(jnp.exp(-jnp.abs(logits))))


def _loss_kernel(p_ref, g_ref, out_ref, acc):
    step = pl.program_id(0)

    @pl.when(step == 0)
    def _init():
        for k in range(6):
            acc[k] = 0.0

    gc = g_ref[:, 0, :]
    mp = (gc > 0.5).astype(jnp.float32)
    mn = (jnp.abs(gc) < 0.5).astype(jnp.float32)
    pc = jax.nn.sigmoid(p_ref[:, 0, :])
    dv = pc - gc
    acc[0] += jnp.sum(dv * dv * (5.0 * mp + mn))
    acc[4] += jnp.sum(mp)

    t2 = 0.0
    for j in range(1, 1 + _NUM_CLASSES):
        t2 += jnp.sum(_bce(p_ref[:, j, :], g_ref[:, j, :]) * mp)
    acc[1] += t2

    mw = mp * g_ref[:, 8, :]
    t3 = 0.0
    for j in (4, 5):
        t3 += jnp.sum(_bce(p_ref[:, j, :], g_ref[:, j, :]) * mw)
    acc[2] += t3

    t4 = 0.0
    for j in (6, 7):
        d = p_ref[:, j, :] - g_ref[:, j, :]
        t4 += jnp.sum(d * d * mw)
    acc[3] += t4

    @pl.when(step == pl.num_programs(0) - 1)
    def _fin():
        npos = jnp.maximum(acc[4], 1.0)
        out_ref[0] = ((acc[0] + acc[2] + acc[3]) / float(_B)
                      + acc[1] / npos)


def kernel(pyolos, gyolos):
    pn = (pyolos.reshape(_B, _CH, _NUM_ANC, _HW)
          .transpose(0, 1, 3, 2)
          .reshape(_B, _CH, _N))
    gt = jnp.transpose(gyolos, (0, 2, 1))
    out = pl.pallas_call(
        _loss_kernel,
        grid=(_B // _IB,),
        in_specs=[
            pl.BlockSpec((_IB, _CH, _N), lambda i: (i, 0, 0)),
            pl.BlockSpec((_IB, 13, _N), lambda i: (i, 0, 0)),
        ],
        out_specs=pl.BlockSpec(memory_space=pltpu.SMEM),
        out_shape=jax.ShapeDtypeStruct((1,), jnp.float32),
        scratch_shapes=[pltpu.SMEM((8,), jnp.float32)],
        compiler_params=pltpu.CompilerParams(
            dimension_semantics=("arbitrary",)),
    )(pn, gt)
    return out[0]
